# baseline (speedup 1.0000x reference)
import jax
import jax.numpy as jnp
from jax import lax
from jax.experimental import pallas as pl
from jax.experimental.pallas import tpu as pltpu
from jax.experimental.pallas import tpu_sc as plsc

_B, _L, _D = 4096, 50, 64
_V = 1000000
_VH = _V // 2
_NW = 32
_BAGS_W = _B // _NW
_CB = 32
_NCHUNK = _BAGS_W // _CB
_ROWS = _CB * _L
_TB = 32768
_TQ = _TB // 4
_SHQ = _TQ.bit_length() - 1
_NBLK = (_V + _TB - 1) // _TB
_VS = _NBLK * _TB


def _repack_body(x_ref, o_ref):
    x = x_ref[...]
    lo = lax.bitcast_convert_type(x[0:32, :], jnp.uint32)
    hi = lax.bitcast_convert_type(x[32:64, :], jnp.uint32)
    w = (((lo + 0x8000) >> 16)
         | ((hi + 0x8000) & jnp.uint32(0xFFFF0000)))
    o_ref[...] = jnp.concatenate(
        [w[:, q * _TQ:(q + 1) * _TQ] for q in range(4)], axis=0).T


def _repack(embT):
    return pl.pallas_call(
        _repack_body,
        grid=(_NBLK,),
        in_specs=[pl.BlockSpec((_D, _TB), lambda j: (0, j))],
        out_specs=pl.BlockSpec((_TQ, 128), lambda j: (j, 0)),
        out_shape=jax.ShapeDtypeStruct((_VS // 4, 128), jnp.uint32),
        compiler_params=pltpu.CompilerParams(
            dimension_semantics=("parallel",)),
    )(embT)


def _pool_body(text_ref, emb_ref, pooled_ref, idx0, idx1, rows0, rows1,
               pool_v, sem0, sem1):
    cid = lax.axis_index("c")
    sid = lax.axis_index("s")
    wid = sid * 2 + cid
    bag0 = wid * _BAGS_W
    idxs, rows, sems = (idx0, idx1), (rows0, rows1), (sem0, sem1)
    handles = {}

    def issue(c):
        p = c % 2
        idx_v = idxs[p]
        tok0 = (bag0 + c * _CB) * _L
        pltpu.sync_copy(text_ref.at[pl.ds(tok0, _ROWS)], idx_v)

        def fix_idx(i, carry2):
            v = idx_v[pl.ds(16 * i, 16)]
            idx_v[pl.ds(16 * i, 16)] = ((v & -_TB) + ((v & (_TQ - 1)) * 4)
                                        + ((v >> _SHQ) & 3))
            return carry2
        lax.fori_loop(0, _ROWS // 16, fix_idx, 0)
        handles[c] = pltpu.async_copy(emb_ref.at[idx_v], rows[p], sems[p])

    issue(0)
    for c in range(_NCHUNK):
        if c + 1 < _NCHUNK:
            issue(c + 1)
        handles[c].wait()
        rows_v = rows[c % 2]

        for b in range(_CB):
            def rbody(r, accs, rows_v=rows_v, b=b):
                row = b * _L + r
                a0, a1, a2, a3 = accs
                w0 = rows_v[row, pl.ds(0, 16)]
                w1 = rows_v[row, pl.ds(16, 16)]
                a0 = a0 + lax.bitcast_convert_type(w0 << 16, jnp.float32)
                a1 = a1 + lax.bitcast_convert_type(w1 << 16, jnp.float32)
                a2 = a2 + lax.bitcast_convert_type(
                    w0 & jnp.uint32(0xFFFF0000), jnp.float32)
                a3 = a3 + lax.bitcast_convert_type(
                    w1 & jnp.uint32(0xFFFF0000), jnp.float32)
                return (a0, a1, a2, a3)
            accs = lax.fori_loop(
                0, _L, rbody,
                tuple(jnp.zeros((16,), jnp.float32) for _ in range(4)))
            out_row = c * _CB + b
            for k in range(4):
                pool_v[out_row, pl.ds(16 * k, 16)] = accs[k] * (1.0 / _L)

    pltpu.sync_copy(pool_v, pooled_ref.at[pl.ds(bag0, _BAGS_W)])


_pool = pl.kernel(
    _pool_body,
    out_type=jax.ShapeDtypeStruct((_B, _D), jnp.float32),
    mesh=plsc.VectorSubcoreMesh(core_axis_name="c", subcore_axis_name="s"),
    compiler_params=pltpu.CompilerParams(use_tc_tiling_on_sc=False),
    scratch_types=[
        pltpu.VMEM((_ROWS,), jnp.int32),
        pltpu.VMEM((_ROWS,), jnp.int32),
        pltpu.VMEM((_ROWS, 32), jnp.uint32),
        pltpu.VMEM((_ROWS, 32), jnp.uint32),
        pltpu.VMEM((_BAGS_W, _D), jnp.float32),
        pltpu.SemaphoreType.DMA,
        pltpu.SemaphoreType.DMA,
    ],
)


def _mlp_body(p_ref, w1t_ref, b1_ref, w2t_ref, b2_ref, o_ref):
    h = jnp.dot(p_ref[...], w1t_ref[...], preferred_element_type=jnp.float32)
    h = jnp.maximum(h + b1_ref[...], 0.0)
    o_ref[...] = (jnp.dot(h, w2t_ref[...], preferred_element_type=jnp.float32)
                  + b2_ref[...])


def kernel(text, offsets, emb, W1, b1, W2, b2):
    del offsets
    staged = _repack(emb.T)
    embL = staged.reshape(_VS, 32)
    pooled = _pool(text, embL)
    ncls = W2.shape[0]
    out = pl.pallas_call(
        _mlp_body,
        out_shape=jax.ShapeDtypeStruct((_B, ncls), jnp.float32),
    )(pooled, W1.T, b1.reshape(1, -1), W2.T, b2.reshape(1, -1))
    return out
